# baseline (speedup 1.0000x reference)
import jax
import jax.numpy as jnp
from jax import lax
from jax.experimental import pallas as pl
from jax.experimental.pallas import tpu as pltpu
from jax.experimental.pallas import tpu_sc as plsc

BATCH = 16384
ENT_DIM = 128
HALF = 64
LANES = 16
NUM_WORKERS = 32
SAMPLES_PER_WORKER = BATCH // NUM_WORKERS
CHUNK = 64
NUM_CHUNKS = SAMPLES_PER_WORKER // CHUNK
GROUPS_PER_CHUNK = CHUNK // LANES


def _body(sample_hbm, ent_hbm, rel_hbm, out_hbm,
          s_buf, hidx_v, ridx_v, tidx_v,
          h_bufs, r_bufs, t_bufs, score_v, sem0, sem1):
    wid = lax.axis_index("s") * 2 + lax.axis_index("c")
    wbase = wid * SAMPLES_PER_WORKER
    iota = lax.broadcasted_iota(jnp.int32, (LANES,), 0)
    last_lane = iota == (LANES - 1)
    sems = (sem0, sem1)

    pltpu.sync_copy(sample_hbm.at[pl.ds(wbase, SAMPLES_PER_WORKER)], s_buf)

    def extract_chunk(c):
        for g in range(GROUPS_PER_CHUNK):
            row = c * CHUNK + g * LANES + iota
            sl = pl.ds(g * LANES, LANES)
            hidx_v[c, sl] = plsc.load_gather(s_buf, [row, jnp.zeros((LANES,), jnp.int32)])
            ridx_v[c, sl] = plsc.load_gather(s_buf, [row, jnp.ones((LANES,), jnp.int32)])
            tidx_v[c, sl] = plsc.load_gather(s_buf, [row, jnp.full((LANES,), 2, jnp.int32)])

    def start_gathers(c):
        par = c % 2
        s = sems[par]
        return (pltpu.async_copy(ent_hbm.at[hidx_v.at[c]], h_bufs[par], s),
                pltpu.async_copy(rel_hbm.at[ridx_v.at[c]], r_bufs[par], s),
                pltpu.async_copy(ent_hbm.at[tidx_v.at[c]], t_bufs[par], s))

    extract_chunk(0)
    inflight = start_gathers(0)
    for c in range(1, NUM_CHUNKS):
        extract_chunk(c)

    for c in range(NUM_CHUNKS):
        par = c % 2
        for cp in inflight:
            cp.wait()
        if c + 1 < NUM_CHUNKS:
            inflight = start_gathers(c + 1)
        h_buf, r_buf, t_buf = h_bufs[par], r_bufs[par], t_bufs[par]

        def sample_body(s, carry):
            acc = jnp.zeros((LANES,), jnp.float32)
            for k in range(HALF // LANES):
                re_sl = pl.ds(k * LANES, LANES)
                im_sl = pl.ds(HALF + k * LANES, LANES)
                rh = h_buf[s, re_sl]
                ih = h_buf[s, im_sl]
                rr = r_buf[s, re_sl]
                ir = r_buf[s, im_sl]
                rt = t_buf[s, re_sl]
                it = t_buf[s, im_sl]
                acc = acc + (rh * rr - ih * ir) * rt + (rh * ir + ih * rr) * it
            cum = plsc.cumsum(acc)
            pos = jnp.full((LANES,), carry + s, jnp.int32)
            plsc.store_scatter(score_v, [pos], cum, mask=last_lane)
            return carry

        lax.fori_loop(0, CHUNK, sample_body, c * CHUNK)

    pltpu.sync_copy(score_v, out_hbm.at[pl.ds(wbase, SAMPLES_PER_WORKER)])


_sc_call = pl.kernel(
    _body,
    out_type=jax.ShapeDtypeStruct((BATCH,), jnp.float32),
    mesh=plsc.VectorSubcoreMesh(core_axis_name="c", subcore_axis_name="s"),
    scratch_types=[
        pltpu.VMEM((SAMPLES_PER_WORKER, 3), jnp.int32),
        pltpu.VMEM((NUM_CHUNKS, CHUNK), jnp.int32),
        pltpu.VMEM((NUM_CHUNKS, CHUNK), jnp.int32),
        pltpu.VMEM((NUM_CHUNKS, CHUNK), jnp.int32),
        (pltpu.VMEM((CHUNK, ENT_DIM), jnp.float32),
         pltpu.VMEM((CHUNK, ENT_DIM), jnp.float32)),
        (pltpu.VMEM((CHUNK, ENT_DIM), jnp.float32),
         pltpu.VMEM((CHUNK, ENT_DIM), jnp.float32)),
        (pltpu.VMEM((CHUNK, ENT_DIM), jnp.float32),
         pltpu.VMEM((CHUNK, ENT_DIM), jnp.float32)),
        pltpu.VMEM((SAMPLES_PER_WORKER,), jnp.float32),
        pltpu.SemaphoreType.DMA,
        pltpu.SemaphoreType.DMA,
    ],
    compiler_params=pltpu.CompilerParams(needs_layout_passes=False),
)


@jax.jit
def kernel(sample, entity_embedding, relation_embedding):
    score = _sc_call(sample, entity_embedding, relation_embedding)
    return score.reshape(BATCH, 1), jnp.zeros((), dtype=jnp.float32)
